# baseline (speedup 1.0000x reference)
import functools

import jax
import jax.numpy as jnp
import numpy as np
from jax import lax
from jax.experimental import pallas as pl
from jax.experimental.pallas import tpu as pltpu
from jax.experimental.pallas import tpu_sc as plsc

_B, _N, _C = 16, 100000, 7
_NUM_POINTS = 20000
_NC, _NS = 2, 16
_NW = _NC * _NS
_TOTAL = _B * _NUM_POINTS
_ROWS_PER_W = _TOTAL // _NW
_CH = 125
_NCH = _ROWS_PER_W // _CH

_IDX_CACHE = None


def _flat_indices():
    global _IDX_CACHE
    if _IDX_CACHE is None:
        with jax.ensure_compile_time_eval():
            base_key = jax.random.key(42)
            rows = []
            for i in range(_B):
                k = jax.random.fold_in(base_key, i)
                rows.append(jax.random.choice(k, _N, shape=(_NUM_POINTS,),
                                              replace=False))
            idx = jnp.stack(rows)
            idx = idx + jnp.arange(_B, dtype=idx.dtype)[:, None] * _N
            _IDX_CACHE = np.asarray(idx, dtype=np.int32).reshape(-1)
    return _IDX_CACHE


def _build_gather():
    mesh = plsc.VectorSubcoreMesh(core_axis_name="c", subcore_axis_name="s")

    @functools.partial(
        pl.kernel,
        out_type=jax.ShapeDtypeStruct((_NW, _NCH, _CH, _C), jnp.float32),
        mesh=mesh,
        scratch_types=[
            pltpu.VMEM((_NCH, _CH), jnp.int32),
            pltpu.VMEM((_NCH, _CH, _C), jnp.float32),
            pltpu.SemaphoreType.DMA,
        ],
        compiler_params=pltpu.CompilerParams(use_tc_tiling_on_sc=False),
    )
    def gather_k(table_hbm, idx_hbm, out_hbm, idx_v, rows_v, sem):
        wid = lax.axis_index("s") * _NC + lax.axis_index("c")
        pltpu.sync_copy(idx_hbm.at[wid], idx_v)

        def chunk(j, _):
            pltpu.async_copy(table_hbm.at[idx_v.at[j]], rows_v.at[j],
                             sem).wait()
            return None

        lax.fori_loop(0, _NCH, chunk, None)
        pltpu.sync_copy(rows_v, out_hbm.at[wid])

    return gather_k


def kernel(points):
    table = points.reshape(_B * _N, _C)
    idx = jnp.asarray(_flat_indices()).reshape(_NW, _NCH, _CH)
    out = _build_gather()(table, idx)
    return out.reshape(_B, _NUM_POINTS, _C)


# speedup vs baseline: 1.0074x; 1.0074x over previous
import functools

import jax
import jax.numpy as jnp
import numpy as np
from jax import lax
from jax.experimental import pallas as pl
from jax.experimental.pallas import tpu as pltpu
from jax.experimental.pallas import tpu_sc as plsc

_B, _N, _C = 16, 100000, 7
_CP = 8
_NUM_POINTS = 20000
_NC, _NS = 2, 16
_NW = _NC * _NS
_TOTAL = _B * _NUM_POINTS
_ROWS_PER_W = _TOTAL // _NW
_CH = 80
_NCH = _ROWS_PER_W // _CH

_IDX_CACHE = None


def _flat_indices():
    global _IDX_CACHE
    if _IDX_CACHE is None:
        with jax.ensure_compile_time_eval():
            base_key = jax.random.key(42)
            rows = []
            for i in range(_B):
                k = jax.random.fold_in(base_key, i)
                rows.append(jax.random.choice(k, _N, shape=(_NUM_POINTS,),
                                              replace=False))
            idx = jnp.stack(rows)
            idx = idx + jnp.arange(_B, dtype=idx.dtype)[:, None] * _N
            _IDX_CACHE = np.asarray(idx, dtype=np.int32).reshape(-1)
    return _IDX_CACHE


def _build_gather():
    mesh = plsc.VectorSubcoreMesh(core_axis_name="c", subcore_axis_name="s")

    @functools.partial(
        pl.kernel,
        out_type=jax.ShapeDtypeStruct((_NW, _NCH, _CH, _CP), jnp.float32),
        mesh=mesh,
        scratch_types=[
            pltpu.VMEM((_NCH, _CH), jnp.int32),
            pltpu.VMEM((_NCH, _CH, _CP), jnp.float32),
            pltpu.SemaphoreType.DMA,
        ],
        compiler_params=pltpu.CompilerParams(use_tc_tiling_on_sc=False),
    )
    def gather_k(table_hbm, idx_hbm, out_hbm, idx_v, rows_v, sem):
        wid = lax.axis_index("s") * _NC + lax.axis_index("c")
        pltpu.sync_copy(idx_hbm.at[wid], idx_v)

        def chunk(j, _):
            pltpu.async_copy(table_hbm.at[idx_v.at[j]], rows_v.at[j],
                             sem).wait()
            return None

        lax.fori_loop(0, _NCH, chunk, None)
        pltpu.sync_copy(rows_v, out_hbm.at[wid])

    return gather_k


def kernel(points):
    padded = jnp.concatenate(
        [points, jnp.zeros((_B, _N, _CP - _C), jnp.float32)], axis=-1)
    table = padded.reshape(_B * _N, _CP)
    idx = jnp.asarray(_flat_indices()).reshape(_NW, _NCH, _CH)
    out = _build_gather()(table, idx)
    return out.reshape(_B, _NUM_POINTS, _CP)[..., :_C]


# speedup vs baseline: 1.0486x; 1.0409x over previous
import functools

import jax
import jax.numpy as jnp
import numpy as np
from jax import lax
from jax.experimental import pallas as pl
from jax.experimental.pallas import tpu as pltpu
from jax.experimental.pallas import tpu_sc as plsc

_B, _N, _C = 16, 100000, 7
_CP = 8
_NUM_POINTS = 20000
_NC, _NS = 2, 16
_NW = _NC * _NS
_TOTAL = _B * _NUM_POINTS
_ROWS_PER_W = _TOTAL // _NW
_CH = 80
_NCH = _ROWS_PER_W // _CH
_GROUP = 5

_IDX_CACHE = None


def _flat_indices():
    global _IDX_CACHE
    if _IDX_CACHE is None:
        with jax.ensure_compile_time_eval():
            base_key = jax.random.key(42)
            rows = []
            for i in range(_B):
                k = jax.random.fold_in(base_key, i)
                rows.append(jax.random.choice(k, _N, shape=(_NUM_POINTS,),
                                              replace=False))
            idx = jnp.stack(rows)
            idx = idx + jnp.arange(_B, dtype=idx.dtype)[:, None] * _N
            _IDX_CACHE = np.asarray(idx, dtype=np.int32).reshape(-1)
    return _IDX_CACHE


def _build_gather():
    mesh = plsc.VectorSubcoreMesh(core_axis_name="c", subcore_axis_name="s")

    @functools.partial(
        pl.kernel,
        out_type=jax.ShapeDtypeStruct((_NW, _NCH, _CH, _CP), jnp.float32),
        mesh=mesh,
        scratch_types=[
            pltpu.VMEM((_NCH, _CH), jnp.int32),
            pltpu.VMEM((_NCH, _CH, _CP), jnp.float32),
            pltpu.SemaphoreType.DMA((_GROUP,)),
        ],
        compiler_params=pltpu.CompilerParams(use_tc_tiling_on_sc=False),
    )
    def gather_k(table_hbm, idx_hbm, out_hbm, idx_v, rows_v, sem):
        wid = lax.axis_index("s") * _NC + lax.axis_index("c")
        pltpu.sync_copy(idx_hbm.at[wid], idx_v)

        def group(g, _):
            copies = [
                pltpu.async_copy(table_hbm.at[idx_v.at[g * _GROUP + u]],
                                 rows_v.at[g * _GROUP + u], sem.at[u])
                for u in range(_GROUP)
            ]
            for c in copies:
                c.wait()
            return None

        lax.fori_loop(0, _NCH // _GROUP, group, None)
        pltpu.sync_copy(rows_v, out_hbm.at[wid])

    return gather_k


def kernel(points):
    padded = jnp.concatenate(
        [points, jnp.zeros((_B, _N, _CP - _C), jnp.float32)], axis=-1)
    table = padded.reshape(_B * _N, _CP)
    idx = jnp.asarray(_flat_indices()).reshape(_NW, _NCH, _CH)
    out = _build_gather()(table, idx)
    return out.reshape(_B, _NUM_POINTS, _CP)[..., :_C]


# speedup vs baseline: 1.3694x; 1.3059x over previous
import functools

import jax
import jax.numpy as jnp
import numpy as np
from jax import lax
from jax.experimental import pallas as pl
from jax.experimental.pallas import tpu as pltpu
from jax.experimental.pallas import tpu_sc as plsc

_B, _N, _C = 16, 100000, 7
_NUM_POINTS = 20000
_NC, _NS = 2, 16
_NW = _NC * _NS
_ROWS_PER_W = _B * _NUM_POINTS // _NW
_R = 2000
_K = _N // _R
_R7 = _R * _C
_OUT_W = _ROWS_PER_W * _C
_SINK = _OUT_W
_OUT_BUF = _OUT_W + 16

_PLAN_CACHE = None


def _sampled_indices():
    with jax.ensure_compile_time_eval():
        base_key = jax.random.key(42)
        rows = []
        for i in range(_B):
            k = jax.random.fold_in(base_key, i)
            rows.append(jax.random.choice(k, _N, shape=(_NUM_POINTS,),
                                          replace=False))
        return np.asarray(jnp.stack(rows), dtype=np.int64)


def _scan_plan():
    global _PLAN_CACHE
    if _PLAN_CACHE is None:
        idx = _sampled_indices()
        per_w = []
        smax = 0
        for w in range(_NW):
            i, h = divmod(w, 2)
            iw = idx[i, h * _ROWS_PER_W:(h + 1) * _ROWS_PER_W]
            k = iw // _R
            lp = (iw % _R) * _C
            dp = np.arange(_ROWS_PER_W, dtype=np.int64) * _C
            counts = np.bincount(k, minlength=_K)
            smax = max(smax, int(counts.max()))
            per_w.append((k, lp, dp))
        s = -(-smax // 16) * 16
        lp_arr = np.zeros((_NW, _K, s), np.int32)
        dp_arr = np.full((_NW, _K, s), _SINK, np.int32)
        for w, (k, lp, dp) in enumerate(per_w):
            order = np.argsort(k, kind="stable")
            k, lp, dp = k[order], lp[order], dp[order]
            pos = 0
            for kk in range(_K):
                n = int(np.searchsorted(k, kk + 1)) - pos
                lp_arr[w, kk, :n] = lp[pos:pos + n]
                dp_arr[w, kk, :n] = dp[pos:pos + n]
                pos += n
        _PLAN_CACHE = (lp_arr, dp_arr, s)
    return _PLAN_CACHE


def _build_scan_kernel(s):
    mesh = plsc.VectorSubcoreMesh(core_axis_name="c", subcore_axis_name="s")

    @functools.partial(
        pl.kernel,
        out_type=jax.ShapeDtypeStruct((_NW, _OUT_W), jnp.float32),
        mesh=mesh,
        scratch_types=[
            pltpu.VMEM((_K, s), jnp.int32),
            pltpu.VMEM((_K, s), jnp.int32),
            pltpu.VMEM((_R7,), jnp.float32),
            pltpu.VMEM((_R7,), jnp.float32),
            pltpu.VMEM((_OUT_BUF,), jnp.float32),
            pltpu.SemaphoreType.DMA,
            pltpu.SemaphoreType.DMA,
        ],
        compiler_params=pltpu.CompilerParams(use_tc_tiling_on_sc=False,
                                             needs_layout_passes=False),
    )
    def scan_k(pts_hbm, lp_hbm, dp_hbm, out_hbm,
               lp_v, dp_v, buf0, buf1, out_v, sem0, sem1):
        wid = lax.axis_index("s") * _NC + lax.axis_index("c")
        inst = wid // 2
        bufs = (buf0, buf1)
        sems = (sem0, sem1)

        def fire(k):
            return pltpu.async_copy(
                pts_hbm.at[inst, pl.ds(k * _R7, _R7)], bufs[k % 2],
                sems[k % 2])

        def process(k):
            buf = bufs[k % 2]

            def body(g, carry):
                lp16 = lp_v[k, pl.ds(g * 16, 16)]
                dp16 = dp_v[k, pl.ds(g * 16, 16)]
                for c in range(_C):
                    v = plsc.load_gather(buf, [lp16 + c])
                    plsc.store_scatter(out_v, [dp16 + c], v)
                return carry

            lax.fori_loop(0, s // 16, body, 0)

        copies = [None] * _K
        copies[0] = fire(0)
        copies[1] = fire(1)
        pltpu.sync_copy(lp_hbm.at[wid], lp_v)
        pltpu.sync_copy(dp_hbm.at[wid], dp_v)
        for k in range(_K):
            copies[k].wait()
            process(k)
            if k + 2 < _K:
                copies[k + 2] = fire(k + 2)
        pltpu.sync_copy(out_v.at[pl.ds(0, _OUT_W)], out_hbm.at[wid])

    return scan_k


def kernel(points):
    lp_arr, dp_arr, s = _scan_plan()
    pts = points.reshape(_B, _N * _C)
    out = _build_scan_kernel(s)(pts, jnp.asarray(lp_arr), jnp.asarray(dp_arr))
    return out.reshape(_B, _NUM_POINTS, _C)
